# baseline (speedup 1.0000x reference)
import jax
import jax.numpy as jnp
from jax import lax
from jax.experimental import pallas as pl
from jax.experimental.pallas import tpu as pltpu
from jax.experimental.pallas import tpu_sc as plsc

_DIM = 64
_SCALE = 8.0
_NW = 32
_L = 16
_BLK = 128


def _gather_t(x_t, table):
    s, n = x_t.shape
    cols_per_w = n // _NW
    nblk = cols_per_w // _BLK
    nit = n // _BLK

    mesh = plsc.VectorSubcoreMesh(core_axis_name="c", subcore_axis_name="s")

    @pl.kernel(
        out_type=jax.ShapeDtypeStruct((s, 8, nit, _DIM // 8, _BLK), jnp.float32),
        mesh=mesh,
        scratch_types=[
            pltpu.VMEM((s, cols_per_w), jnp.int32),
            [pltpu.VMEM((_BLK, _DIM), jnp.float32) for _ in range(2)],
            [pltpu.VMEM((_DIM, _BLK), jnp.float32) for _ in range(2)],
            [pltpu.SemaphoreType.DMA for _ in range(2)],
            [pltpu.SemaphoreType.DMA for _ in range(2)],
        ],
        compiler_params=pltpu.CompilerParams(
            use_tc_tiling_on_sc=False, needs_layout_passes=False
        ),
    )
    def k(x_hbm, t_hbm, out_hbm, idx_v, gbufs, obufs, gsems, osems):
        w = lax.axis_index("s") * 2 + lax.axis_index("c")
        i0 = w * cols_per_w
        ib0 = w * nblk
        iota = lax.iota(jnp.int32, _L)
        row_idx = [iota + (m * _L) for m in range(_BLK // _L)]

        pltpu.sync_copy(x_hbm.at[:, pl.ds(i0, cols_per_w)], idx_v)

        nsteps = s * nblk

        def start_gather(t, b):
            j = t // nblk
            ib = lax.rem(t, nblk)
            pltpu.async_copy(
                t_hbm.at[idx_v.at[j, pl.ds(ib * _BLK, _BLK)]], gbufs[b], gsems[b]
            )

        def wait_gather(b):
            pltpu.make_async_copy(
                t_hbm.at[idx_v.at[0, pl.ds(0, _BLK)]], gbufs[b], gsems[b]
            ).wait()

        def start_out(t, b):
            j = t // nblk
            ib = lax.rem(t, nblk)
            for dt in range(8):
                pltpu.async_copy(
                    obufs[b].at[pl.ds(dt * 8, 8)],
                    out_hbm.at[j, dt, ib0 + ib],
                    osems[b],
                )

        def wait_out(b):
            for _dt in range(8):
                pltpu.make_async_copy(
                    obufs[b].at[pl.ds(0, 8)], out_hbm.at[0, 0, 0], osems[b]
                ).wait()

        def transpose(b):
            @plsc.parallel_loop(0, _BLK, unroll=16)
            def _(i):
                iv = jnp.full((_L,), 0, jnp.int32) + i
                for kk in range(_DIM // _L):
                    vals = gbufs[b][i, pl.ds(kk * _L, _L)] * _SCALE
                    plsc.store_scatter(obufs[b], [row_idx[kk], iv], vals)

        def step(t, carry):
            b = lax.rem(t, 2)

            def do(bb):
                @pl.when(b == bb)
                def _():
                    @pl.when(t >= 2)
                    def _():
                        wait_out(bb)

                    wait_gather(bb)
                    transpose(bb)
                    start_out(t, bb)

                    @pl.when(t + 2 < nsteps)
                    def _():
                        start_gather(t + 2, bb)

            do(0)
            do(1)
            return carry

        start_gather(0, 0)
        start_gather(1, 1)
        lax.fori_loop(0, nsteps, step, 0)
        wait_out(0)
        wait_out(1)

    return k(x_t, table)


def kernel(x, table):
    n, s = x.shape
    x_t = jnp.transpose(x).astype(jnp.int32)
    y = _gather_t(x_t, table)
    return jnp.transpose(y, (2, 4, 0, 1, 3)).reshape(n, s, _DIM)


# speedup vs baseline: 1.0104x; 1.0104x over previous
import jax
import jax.numpy as jnp
from jax import lax
from jax.experimental import pallas as pl
from jax.experimental.pallas import tpu as pltpu
from jax.experimental.pallas import tpu_sc as plsc

_DIM = 64
_SCALE = 8.0
_NW = 32
_L = 16
_BLK = 128


def _gather_t(x_t, table):
    s, n = x_t.shape
    cols_per_w = n // _NW
    nblk = cols_per_w // _BLK
    nit = n // _BLK

    mesh = plsc.VectorSubcoreMesh(core_axis_name="c", subcore_axis_name="s")

    @pl.kernel(
        out_type=jax.ShapeDtypeStruct((s, 8, nit, _DIM // 8, _BLK), jnp.float32),
        mesh=mesh,
        scratch_types=[
            pltpu.VMEM((s, cols_per_w), jnp.int32),
            [pltpu.VMEM((_BLK, _DIM), jnp.float32) for _ in range(2)],
            [pltpu.VMEM((_DIM, _BLK), jnp.float32) for _ in range(2)],
            [pltpu.SemaphoreType.DMA for _ in range(2)],
            [pltpu.SemaphoreType.DMA for _ in range(2)],
        ],
        compiler_params=pltpu.CompilerParams(
            use_tc_tiling_on_sc=False, needs_layout_passes=False
        ),
    )
    def k(x_hbm, t_hbm, out_hbm, idx_v, gbufs, obufs, gsems, osems):
        w = lax.axis_index("s") * 2 + lax.axis_index("c")
        i0 = w * cols_per_w
        ib0 = w * nblk
        iota = lax.iota(jnp.int32, _L)
        row_idx = [iota + (m * _L) for m in range(_BLK // _L)]

        pltpu.sync_copy(x_hbm.at[:, pl.ds(i0, cols_per_w)], idx_v)

        nsteps = s * nblk

        def start_gather(t, b):
            j = t // nblk
            ib = lax.rem(t, nblk)
            pltpu.async_copy(
                t_hbm.at[idx_v.at[j, pl.ds(ib * _BLK, _BLK)]], gbufs[b], gsems[b]
            )

        def wait_gather(b):
            pltpu.make_async_copy(
                t_hbm.at[idx_v.at[0, pl.ds(0, _BLK)]], gbufs[b], gsems[b]
            ).wait()

        def start_out(t, b):
            j = t // nblk
            ib = lax.rem(t, nblk)
            for dt in range(8):
                pltpu.async_copy(
                    obufs[b].at[pl.ds(dt * 8, 8)],
                    out_hbm.at[j, dt, ib0 + ib],
                    osems[b],
                )

        def wait_out(b):
            for _dt in range(8):
                pltpu.make_async_copy(
                    obufs[b].at[pl.ds(0, 8)], out_hbm.at[0, 0, 0], osems[b]
                ).wait()

        def transpose(b):
            @plsc.parallel_loop(0, _BLK, unroll=8)
            def _(i):
                iv = jnp.full((_L,), 0, jnp.int32) + i
                for kk in range(_DIM // _L):
                    vals = gbufs[b][i, pl.ds(kk * _L, _L)] * _SCALE
                    plsc.store_scatter(obufs[b], [row_idx[kk], iv], vals)

        def step(t, carry):
            b = lax.rem(t, 2)

            def do(bb):
                @pl.when(b == bb)
                def _():
                    @pl.when(t >= 2)
                    def _():
                        wait_out(bb)

                    wait_gather(bb)
                    transpose(bb)
                    start_out(t, bb)

                    @pl.when(t + 2 < nsteps)
                    def _():
                        start_gather(t + 2, bb)

            do(0)
            do(1)
            return carry

        start_gather(0, 0)
        start_gather(1, 1)
        lax.fori_loop(0, nsteps, step, 0)
        wait_out(0)
        wait_out(1)

    return k(x_t, table)


def kernel(x, table):
    n, s = x.shape
    x_t = jnp.transpose(x).astype(jnp.int32)
    y = _gather_t(x_t, table)
    return jnp.transpose(y, (2, 4, 0, 1, 3)).reshape(n, s, _DIM)


# speedup vs baseline: 1.0134x; 1.0029x over previous
import jax
import jax.numpy as jnp
from jax import lax
from jax.experimental import pallas as pl
from jax.experimental.pallas import tpu as pltpu
from jax.experimental.pallas import tpu_sc as plsc

_DIM = 64
_SCALE = 8.0
_NW = 32
_L = 16
_BLK = 128


def _gather_t(x_t, table):
    s, n = x_t.shape
    cols_per_w = n // _NW
    nblk = cols_per_w // _BLK
    nit = n // _BLK

    mesh = plsc.VectorSubcoreMesh(core_axis_name="c", subcore_axis_name="s")

    @pl.kernel(
        out_type=jax.ShapeDtypeStruct((s, 8, nit, _DIM // 8, _BLK), jnp.float32),
        mesh=mesh,
        scratch_types=[
            pltpu.VMEM((s, cols_per_w), jnp.int32),
            [pltpu.VMEM((_BLK, _DIM), jnp.float32) for _ in range(2)],
            [pltpu.VMEM((8, _DIM // 8, _BLK), jnp.float32) for _ in range(2)],
            [pltpu.SemaphoreType.DMA for _ in range(2)],
            [pltpu.SemaphoreType.DMA for _ in range(2)],
        ],
        compiler_params=pltpu.CompilerParams(
            use_tc_tiling_on_sc=False, needs_layout_passes=False
        ),
    )
    def k(x_hbm, t_hbm, out_hbm, idx_v, gbufs, obufs, gsems, osems):
        w = lax.axis_index("s") * 2 + lax.axis_index("c")
        i0 = w * cols_per_w
        ib0 = w * nblk
        iota = lax.iota(jnp.int32, _L)
        row_idx = [iota + (m * _L) for m in range(_BLK // _L)]

        pltpu.sync_copy(x_hbm.at[:, pl.ds(i0, cols_per_w)], idx_v)

        nsteps = s * nblk

        def start_gather(t, b):
            j = t // nblk
            ib = lax.rem(t, nblk)
            pltpu.async_copy(
                t_hbm.at[idx_v.at[j, pl.ds(ib * _BLK, _BLK)]], gbufs[b], gsems[b]
            )

        def wait_gather(b):
            pltpu.make_async_copy(
                t_hbm.at[idx_v.at[0, pl.ds(0, _BLK)]], gbufs[b], gsems[b]
            ).wait()

        def start_out(t, b):
            j = t // nblk
            ib = lax.rem(t, nblk)
            for dt in range(8):
                pltpu.async_copy(
                    obufs[b].at[dt],
                    out_hbm.at[j, dt, ib0 + ib],
                    osems[b],
                )

        def wait_out(b):
            pltpu.make_async_copy(
                obufs[b], out_hbm.at[0, 0, pl.ds(0, 8)], osems[b]
            ).wait()

        scat_dt = [(iota + (kk * _L)) // 8 for kk in range(_DIM // _L)]
        scat_d8 = [lax.rem(iota + (kk * _L), 8) for kk in range(_DIM // _L)]

        def transpose(b):
            @plsc.parallel_loop(0, _BLK, unroll=8)
            def _(i):
                iv = jnp.full((_L,), 0, jnp.int32) + i
                for kk in range(_DIM // _L):
                    vals = gbufs[b][i, pl.ds(kk * _L, _L)] * _SCALE
                    plsc.store_scatter(
                        obufs[b], [scat_dt[kk], scat_d8[kk], iv], vals
                    )

        def step(t, carry):
            b = lax.rem(t, 2)

            def do(bb):
                @pl.when(b == bb)
                def _():
                    @pl.when(t >= 2)
                    def _():
                        wait_out(bb)

                    wait_gather(bb)
                    transpose(bb)
                    start_out(t, bb)

                    @pl.when(t + 2 < nsteps)
                    def _():
                        start_gather(t + 2, bb)

            do(0)
            do(1)
            return carry

        start_gather(0, 0)
        start_gather(1, 1)
        lax.fori_loop(0, nsteps, step, 0)
        wait_out(0)
        wait_out(1)

    return k(x_t, table)


def kernel(x, table):
    n, s = x.shape
    x_t = jnp.transpose(x).astype(jnp.int32)
    y = _gather_t(x_t, table)
    return jnp.transpose(y, (2, 4, 0, 1, 3)).reshape(n, s, _DIM)


# speedup vs baseline: 1.6890x; 1.6667x over previous
import jax
import jax.numpy as jnp
from jax import lax
from jax.experimental import pallas as pl
from jax.experimental.pallas import tpu as pltpu
from jax.experimental.pallas import tpu_sc as plsc

_DIM = 64
_SCALE = 8.0
_NW = 32
_L = 16
_BLK = 128


def _gather_t(x_t, table):
    s, n = x_t.shape
    cols_per_w = n // _NW
    nblk = cols_per_w // _BLK
    nit = n // _BLK

    mesh = plsc.VectorSubcoreMesh(core_axis_name="c", subcore_axis_name="s")

    @pl.kernel(
        out_type=jax.ShapeDtypeStruct((s, 8, nit, _DIM // 8, _BLK), jnp.float32),
        mesh=mesh,
        scratch_types=[
            pltpu.VMEM((s, cols_per_w), jnp.int32),
            [pltpu.VMEM((_BLK, _DIM), jnp.float32) for _ in range(2)],
            [pltpu.VMEM((_DIM, _BLK + 1), jnp.float32) for _ in range(2)],
            [pltpu.SemaphoreType.DMA for _ in range(2)],
            [pltpu.SemaphoreType.DMA for _ in range(2)],
        ],
        compiler_params=pltpu.CompilerParams(
            use_tc_tiling_on_sc=False, needs_layout_passes=False
        ),
    )
    def k(x_hbm, t_hbm, out_hbm, idx_v, gbufs, obufs, gsems, osems):
        w = lax.axis_index("s") * 2 + lax.axis_index("c")
        i0 = w * cols_per_w
        ib0 = w * nblk
        iota = lax.iota(jnp.int32, _L)
        row_idx = [iota + (m * _L) for m in range(_BLK // _L)]

        pltpu.sync_copy(x_hbm.at[:, pl.ds(i0, cols_per_w)], idx_v)

        nsteps = s * nblk

        def start_gather(t, b):
            j = t // nblk
            ib = lax.rem(t, nblk)
            pltpu.async_copy(
                t_hbm.at[idx_v.at[j, pl.ds(ib * _BLK, _BLK)]], gbufs[b], gsems[b]
            )

        def wait_gather(b):
            pltpu.make_async_copy(
                t_hbm.at[idx_v.at[0, pl.ds(0, _BLK)]], gbufs[b], gsems[b]
            ).wait()

        def start_out(t, b):
            j = t // nblk
            ib = lax.rem(t, nblk)
            for dt in range(8):
                pltpu.async_copy(
                    obufs[b].at[pl.ds(dt * 8, 8), pl.ds(0, _BLK)],
                    out_hbm.at[j, dt, ib0 + ib],
                    osems[b],
                )

        def wait_out(b):
            for _dt in range(8):
                pltpu.make_async_copy(
                    obufs[b].at[pl.ds(0, 8), pl.ds(0, _BLK)],
                    out_hbm.at[0, 0, 0],
                    osems[b],
                ).wait()

        scat_dt = [(iota + (kk * _L)) // 8 for kk in range(_DIM // _L)]
        scat_d8 = [lax.rem(iota + (kk * _L), 8) for kk in range(_DIM // _L)]

        def transpose(b):
            @plsc.parallel_loop(0, _BLK, unroll=8)
            def _(i):
                iv = jnp.full((_L,), 0, jnp.int32) + i
                for kk in range(_DIM // _L):
                    vals = gbufs[b][i, pl.ds(kk * _L, _L)] * _SCALE
                    plsc.store_scatter(obufs[b], [row_idx[kk], iv], vals)

        def step(t, carry):
            b = lax.rem(t, 2)

            def do(bb):
                @pl.when(b == bb)
                def _():
                    @pl.when(t >= 2)
                    def _():
                        wait_out(bb)

                    wait_gather(bb)
                    transpose(bb)
                    start_out(t, bb)

                    @pl.when(t + 2 < nsteps)
                    def _():
                        start_gather(t + 2, bb)

            do(0)
            do(1)
            return carry

        start_gather(0, 0)
        start_gather(1, 1)
        lax.fori_loop(0, nsteps, step, 0)
        wait_out(0)
        wait_out(1)

    return k(x_t, table)


def kernel(x, table):
    n, s = x.shape
    x_t = jnp.transpose(x).astype(jnp.int32)
    y = _gather_t(x_t, table)
    return jnp.transpose(y, (2, 4, 0, 1, 3)).reshape(n, s, _DIM)
